# Initial kernel scaffold; baseline (speedup 1.0000x reference)
import jax
import jax.numpy as jnp
from jax.experimental import pallas as pl


def kernel(outputs, labels):
    raise NotImplementedError("write your pallas kernel here")


# speedup vs baseline: 1.3133x; 1.3133x over previous
import functools

import jax
import jax.numpy as jnp
import numpy as np
from jax import lax
from jax.experimental import pallas as pl
from jax.experimental.pallas import tpu as pltpu
from jax.experimental.pallas import tpu_sc as plsc

B = 16384
C = 1000
SMOOTHING = 0.1
CONF = 1.0 - SMOOTHING
OFF = SMOOTHING / (C - 1)
KLSE = CONF - OFF + OFF * C
ENT = CONF * float(np.log(CONF)) + (C - 1) * OFF * float(np.log(OFF))


ROWS_PER_BLOCK = 512


def _dense_body(x_ref, acc_ref):
    i = pl.program_id(0)
    x = x_ref[...]
    m = jnp.max(x, axis=1, keepdims=True)
    lse = m + jnp.log(jnp.sum(jnp.exp(x - m), axis=1, keepdims=True))
    partial = jnp.reshape(OFF * jnp.sum(x) - KLSE * jnp.sum(lse), (1, 1))

    @pl.when(i == 0)
    def _():
        acc_ref[...] = jnp.zeros((1, 1), jnp.float32)

    acc_ref[...] += partial


_dense_call = pl.pallas_call(
    _dense_body,
    grid=(B // ROWS_PER_BLOCK,),
    in_specs=[pl.BlockSpec((ROWS_PER_BLOCK, C), lambda i: (i, 0))],
    out_specs=pl.BlockSpec((1, 1), lambda i: (0, 0)),
    out_shape=jax.ShapeDtypeStruct((1, 1), jnp.float32),
)


_INFO = plsc.get_sparse_core_info()
_NC = _INFO.num_cores
_NS = _INFO.num_subcores
_NW = _NC * _NS
_W = B // _NW
_CHUNK = 128

_sc_mesh = plsc.VectorSubcoreMesh(core_axis_name="c", subcore_axis_name="s")


@functools.partial(
    pl.kernel,
    out_type=jax.ShapeDtypeStruct((_NW, 16), jnp.float32),
    mesh=_sc_mesh,
    scratch_types=[
        pltpu.VMEM((_W,), jnp.int32),
        pltpu.VMEM((_W // _CHUNK, _CHUNK), jnp.int32),
        pltpu.VMEM((_W,), jnp.float32),
        pltpu.VMEM((16,), jnp.float32),
        pltpu.SemaphoreType.DMA,
    ],
)
def _sc_gather(flat_hbm, labels_hbm, out_hbm, lab_v, idx_v, vals_v, out_v, sem):
    c = lax.axis_index("c")
    s = lax.axis_index("s")
    wid = s * _NC + c
    base = wid * _W
    pltpu.sync_copy(labels_hbm.at[pl.ds(base, _W)], lab_v)
    for j in range(_W // 16):
        lab = lab_v[pl.ds(j * 16, 16)]
        flat = (base + j * 16 + lax.iota(jnp.int32, 16)) * C + lab
        idx_v[j * 16 // _CHUNK, pl.ds((j * 16) % _CHUNK, 16)] = flat
    copies = [
        pltpu.async_copy(
            flat_hbm.at[idx_v.at[q]], vals_v.at[pl.ds(q * _CHUNK, _CHUNK)], sem
        )
        for q in range(_W // _CHUNK)
    ]
    for cp in copies:
        cp.wait()
    acc = jnp.zeros((16,), jnp.float32)
    for j in range(_W // 16):
        acc = acc + vals_v[pl.ds(j * 16, 16)]
    out_v[...] = acc
    pltpu.sync_copy(out_v, out_hbm.at[wid])


@jax.jit
def kernel(outputs, labels):
    flat = jnp.reshape(outputs, (B * C,))
    g_parts = _sc_gather(flat, labels.astype(jnp.int32))
    acc = _dense_call(outputs)[0, 0]
    g = jnp.sum(g_parts)
    return ENT - (acc + (CONF - OFF) * g) / B


# speedup vs baseline: 2.5335x; 1.9291x over previous
import jax
import jax.numpy as jnp
import numpy as np
from jax import lax
from jax.experimental import pallas as pl

B = 16384
C = 1000
SMOOTHING = 0.1
CONF = 1.0 - SMOOTHING
OFF = SMOOTHING / (C - 1)
KLSE = CONF - OFF + OFF * C
ENT = CONF * float(np.log(CONF)) + (C - 1) * OFF * float(np.log(OFF))

ROWS_PER_BLOCK = 512


def _dense_body(x_ref, lab_ref, acc_ref):
    i = pl.program_id(0)
    x = x_ref[...]
    lab = lab_ref[0, 0, :]
    m = jnp.max(x, axis=1, keepdims=True)
    lse = m + jnp.log(jnp.sum(jnp.exp(x - m), axis=1, keepdims=True))
    col = lax.broadcasted_iota(jnp.int32, (ROWS_PER_BLOCK, C), 1)
    g = jnp.sum(jnp.where(col == lab[:, None], x, 0.0))
    partial = jnp.reshape(
        OFF * jnp.sum(x) - KLSE * jnp.sum(lse) + (CONF - OFF) * g, (1, 1)
    )

    @pl.when(i == 0)
    def _():
        acc_ref[...] = jnp.zeros((1, 1), jnp.float32)

    acc_ref[...] += partial


_dense_call = pl.pallas_call(
    _dense_body,
    grid=(B // ROWS_PER_BLOCK,),
    in_specs=[
        pl.BlockSpec((ROWS_PER_BLOCK, C), lambda i: (i, 0)),
        pl.BlockSpec((1, 1, ROWS_PER_BLOCK), lambda i: (i, 0, 0)),
    ],
    out_specs=pl.BlockSpec((1, 1), lambda i: (0, 0)),
    out_shape=jax.ShapeDtypeStruct((1, 1), jnp.float32),
)


@jax.jit
def kernel(outputs, labels):
    lab3 = jnp.reshape(labels.astype(jnp.int32), (B // ROWS_PER_BLOCK, 1, ROWS_PER_BLOCK))
    acc = _dense_call(outputs, lab3)[0, 0]
    return ENT - acc / B


# speedup vs baseline: 2.7937x; 1.1027x over previous
import jax
import jax.numpy as jnp
import numpy as np
from jax import lax
from jax.experimental import pallas as pl

B = 16384
C = 1000
SMOOTHING = 0.1
CONF = 1.0 - SMOOTHING
OFF = SMOOTHING / (C - 1)
KLSE = CONF - OFF + OFF * C
ENT = CONF * float(np.log(CONF)) + (C - 1) * OFF * float(np.log(OFF))

ROWS_PER_BLOCK = 1024


def _dense_body(x_ref, lab_ref, acc_ref):
    i = pl.program_id(0)
    x = x_ref[...]
    lab = lab_ref[0, 0, :]
    m = jnp.max(x, axis=1, keepdims=True)
    lse = m + jnp.log(jnp.sum(jnp.exp(x - m), axis=1, keepdims=True))
    col = lax.broadcasted_iota(jnp.int32, (ROWS_PER_BLOCK, C), 1)
    g = jnp.sum(jnp.where(col == lab[:, None], x, 0.0))
    partial = jnp.reshape(
        OFF * jnp.sum(x) - KLSE * jnp.sum(lse) + (CONF - OFF) * g, (1, 1)
    )

    @pl.when(i == 0)
    def _():
        acc_ref[...] = jnp.zeros((1, 1), jnp.float32)

    acc_ref[...] += partial


_dense_call = pl.pallas_call(
    _dense_body,
    grid=(B // ROWS_PER_BLOCK,),
    in_specs=[
        pl.BlockSpec((ROWS_PER_BLOCK, C), lambda i: (i, 0)),
        pl.BlockSpec((1, 1, ROWS_PER_BLOCK), lambda i: (i, 0, 0)),
    ],
    out_specs=pl.BlockSpec((1, 1), lambda i: (0, 0)),
    out_shape=jax.ShapeDtypeStruct((1, 1), jnp.float32),
)


@jax.jit
def kernel(outputs, labels):
    lab3 = jnp.reshape(labels.astype(jnp.int32), (B // ROWS_PER_BLOCK, 1, ROWS_PER_BLOCK))
    acc = _dense_call(outputs, lab3)[0, 0]
    return ENT - acc / B


# speedup vs baseline: 2.8880x; 1.0338x over previous
import jax
import jax.numpy as jnp
import numpy as np
from jax import lax
from jax.experimental import pallas as pl

B = 16384
C = 1000
SMOOTHING = 0.1
CONF = 1.0 - SMOOTHING
OFF = SMOOTHING / (C - 1)
KLSE = CONF - OFF + OFF * C
ENT = CONF * float(np.log(CONF)) + (C - 1) * OFF * float(np.log(OFF))

ROWS_PER_BLOCK = 2048


def _dense_body(x_ref, lab_ref, acc_ref):
    i = pl.program_id(0)
    x = x_ref[...]
    lab = lab_ref[0, 0, :]
    m = jnp.max(x, axis=1, keepdims=True)
    lse = m + jnp.log(jnp.sum(jnp.exp(x - m), axis=1, keepdims=True))
    col = lax.broadcasted_iota(jnp.int32, (ROWS_PER_BLOCK, C), 1)
    g = jnp.sum(jnp.where(col == lab[:, None], x, 0.0))
    partial = jnp.reshape(
        OFF * jnp.sum(x) - KLSE * jnp.sum(lse) + (CONF - OFF) * g, (1, 1)
    )

    @pl.when(i == 0)
    def _():
        acc_ref[...] = jnp.zeros((1, 1), jnp.float32)

    acc_ref[...] += partial


_dense_call = pl.pallas_call(
    _dense_body,
    grid=(B // ROWS_PER_BLOCK,),
    in_specs=[
        pl.BlockSpec((ROWS_PER_BLOCK, C), lambda i: (i, 0)),
        pl.BlockSpec((1, 1, ROWS_PER_BLOCK), lambda i: (i, 0, 0)),
    ],
    out_specs=pl.BlockSpec((1, 1), lambda i: (0, 0)),
    out_shape=jax.ShapeDtypeStruct((1, 1), jnp.float32),
)


@jax.jit
def kernel(outputs, labels):
    lab3 = jnp.reshape(labels.astype(jnp.int32), (B // ROWS_PER_BLOCK, 1, ROWS_PER_BLOCK))
    acc = _dense_call(outputs, lab3)[0, 0]
    return ENT - acc / B
